# Initial kernel scaffold; baseline (speedup 1.0000x reference)
import jax
import jax.numpy as jnp
from jax.experimental import pallas as pl


def kernel(variables, lits, Wn, bn, W1v, b1v, W2v, b2v, W1c, b1c, W2c, b2c, Wz, Uz, Wr, Ur, W, U):
    raise NotImplementedError("write your pallas kernel here")


# speedup vs baseline: 2.9706x; 2.9706x over previous
import functools

import jax
import jax.numpy as jnp
from jax import lax
from jax.experimental import pallas as pl
from jax.experimental.pallas import tpu as pltpu
from jax.experimental.pallas import tpu_sc as plsc

_N = 10000
_D = 256
_C = 4
_V = 4

_BN = 1000
_G = 32
_NBLK = (_C * _N) // _G
_NW = 32
_BPW = -(-_NBLK // _NW)


def _dot_t(x, w):
    return lax.dot_general(x, w, (((1,), (1,)), ((), ())),
                           preferred_element_type=jnp.float32)


def _combine(x, w1, b1, w2, b2):
    y = jax.nn.sigmoid(_dot_t(x, w1) + b1) + (_dot_t(x, w2) + b2)
    nrm = jnp.sqrt(jnp.sum(y * y, axis=-1, keepdims=True))
    return y / (nrm + 1e-8)


def _table_body(v_ref, wn_ref, bn_ref, w1_ref, b1_ref, w2_ref, b2_ref,
                out_ref):
    v = v_ref[...]
    nv = _dot_t(v, wn_ref[...]) + bn_ref[...]
    w1, b1 = w1_ref[...], b1_ref[...]
    w2, b2 = w2_ref[...], b2_ref[...]
    out_ref[0] = _combine(v, w1, b1, w2, b2)
    out_ref[1] = _combine(nv, w1, b1, w2, b2)


def _build_table(variables, wn, bn, w1, b1, w2, b2):
    full = pl.BlockSpec((_D, _D), lambda i: (0, 0))
    row = pl.BlockSpec((1, _D), lambda i: (0, 0))
    return pl.pallas_call(
        _table_body,
        grid=(_N // _BN,),
        in_specs=[pl.BlockSpec((_BN, _D), lambda i: (i, 0)),
                  full, row, full, row, full, row],
        out_specs=pl.BlockSpec((2, _BN, _D), lambda i: (0, i, 0)),
        out_shape=jax.ShapeDtypeStruct((2, _N, _D), jnp.float32),
    )(variables, wn, bn.reshape(1, _D), w1, b1.reshape(1, _D),
      w2, b2.reshape(1, _D))


def _sc_body(table, idx, out, idx_v, rows_v, acc_v, sem):
    wid = lax.axis_index("s") * 2 + lax.axis_index("c")

    def block(b, carry):
        t = wid * _BPW + b

        @pl.when(t < _NBLK)
        def _():
            pltpu.sync_copy(idx.at[pl.ds(t * _G * _V, _G * _V)], idx_v)
            pltpu.async_copy(table.at[idx_v], rows_v, sem).wait()

            def clause(g, c2):
                for ch in range(_D // 16):
                    s = pl.ds(ch * 16, 16)
                    acc_v[g, s] = (rows_v[_V * g, s] + rows_v[_V * g + 1, s]
                                   + rows_v[_V * g + 2, s]
                                   + rows_v[_V * g + 3, s])
                return c2

            lax.fori_loop(0, _G, clause, 0)
            pltpu.sync_copy(acc_v, out.at[pl.ds(t * _G, _G)])

        return carry

    lax.fori_loop(0, _BPW, block, 0)


@functools.cache
def _sc_gather_sum_fn():
    mesh = plsc.VectorSubcoreMesh(core_axis_name="c", subcore_axis_name="s")
    return pl.kernel(
        _sc_body,
        mesh=mesh,
        out_type=jax.ShapeDtypeStruct((_C * _N, _D), jnp.float32),
        scratch_types=[
            pltpu.VMEM((_G * _V,), jnp.int32),
            pltpu.VMEM((_G * _V, _D), jnp.float32),
            pltpu.VMEM((_G, _D), jnp.float32),
            pltpu.SemaphoreType.DMA,
        ],
    )


def _update_body(ce_ref, v_ref, w1_ref, b1_ref, w2_ref, b2_ref,
                 wz_ref, uz_ref, wr_ref, ur_ref, w_ref, u_ref, out_ref):
    w1, b1 = w1_ref[...], b1_ref[...]
    w2, b2 = w2_ref[...], b2_ref[...]
    av = _combine(ce_ref[0], w1, b1, w2, b2)
    for c in range(1, _C):
        av = av + _combine(ce_ref[c], w1, b1, w2, b2)
    x = v_ref[...]
    z = jax.nn.sigmoid(_dot_t(av, wz_ref[...]) + _dot_t(x, uz_ref[...]))
    r = jax.nn.sigmoid(_dot_t(av, wr_ref[...]) + _dot_t(x, ur_ref[...]))
    h_t = jnp.tanh(_dot_t(av, w_ref[...]) + _dot_t(r * x, u_ref[...]))
    out_ref[...] = (1.0 - z) * x + z * h_t


def _update(ce, variables, w1, b1, w2, b2, wz, uz, wr, ur, w, u):
    full = pl.BlockSpec((_D, _D), lambda i: (0, 0))
    row = pl.BlockSpec((1, _D), lambda i: (0, 0))
    return pl.pallas_call(
        _update_body,
        grid=(_N // _BN,),
        in_specs=[pl.BlockSpec((_C, _BN, _D), lambda i: (0, i, 0)),
                  pl.BlockSpec((_BN, _D), lambda i: (i, 0)),
                  full, row, full, row, full, full, full, full, full, full],
        out_specs=pl.BlockSpec((_BN, _D), lambda i: (i, 0)),
        out_shape=jax.ShapeDtypeStruct((_N, _D), jnp.float32),
    )(ce, variables, w1, b1.reshape(1, _D), w2, b2.reshape(1, _D),
      wz, uz, wr, ur, w, u)


def kernel(variables, lits, Wn, bn, W1v, b1v, W2v, b2v, W1c, b1c, W2c, b2c,
           Wz, Uz, Wr, Ur, W, U):
    idx_flat = jnp.transpose(lits.astype(jnp.int32), (1, 0, 2)).reshape(-1)
    y_table = _build_table(variables, Wn, bn, W1v, b1v, W2v, b2v)
    ce = _sc_gather_sum_fn()(y_table.reshape(2 * _N, _D), idx_flat)
    return _update(ce.reshape(_C, _N, _D), variables,
                   W1c, b1c, W2c, b2c, Wz, Uz, Wr, Ur, W, U)
